# baseline (speedup 1.0000x reference)
import jax
import jax.numpy as jnp
from jax.experimental import pallas as pl
from jax.experimental.pallas import tpu as pltpu
from jax.experimental.pallas import tpu_sc as plsc

EPS = 1e-3

_GATHER_WINDOW = 256
_VBLK = 1000
_CHUNKS = 4


def _sc_gather(tok_emb, idx2d, n, d):
    mesh = plsc.VectorSubcoreMesh(core_axis_name="core",
                                  subcore_axis_name="subcore")

    @pl.kernel(out_type=jax.ShapeDtypeStruct((n, d), tok_emb.dtype),
               mesh=mesh)
    def gather_kernel(x_hbm, i_hbm, o_hbm):
        def body(i_vmem, o_vmem):
            pltpu.sync_copy(x_hbm.at[i_vmem.at[0]], o_vmem)

        pltpu.emit_pipeline(
            body,
            grid=(n // _GATHER_WINDOW,),
            in_specs=[pl.BlockSpec((1, _GATHER_WINDOW),
                                   index_map=lambda i: (0, i))],
            out_specs=[pl.BlockSpec((_GATHER_WINDOW, d),
                                    index_map=lambda i: (i, 0))],
            core_axis_name=("core", "subcore"),
            dimension_semantics=(pltpu.PARALLEL,),
        )(i_hbm, o_hbm)

    return gather_kernel(tok_emb, idx2d)


def _dense_body(x_ref, pos_ref, gamma_ref, beta_ref, wt_ref, b_ref, o_ref,
                xn_ref):
    d = pos_ref.shape[2]
    vj = pl.program_id(1)

    @pl.when(vj == 0)
    def _():
        x = x_ref[:, :d] + pos_ref[0]
        mean = jnp.mean(x, axis=1, keepdims=True)
        xc = x - mean
        var = jnp.mean(xc * xc, axis=1, keepdims=True)
        xn = xc * jax.lax.rsqrt(var + EPS)
        xn = xn * gamma_ref[...] + beta_ref[...]
        xn_ref[...] = xn.astype(jnp.bfloat16)

    logits = jax.lax.dot_general(
        wt_ref[...], xn_ref[...],
        (((1,), (1,)), ((), ())),
        preferred_element_type=jnp.float32,
    ) + b_ref[...]
    o_ref[...] = logits[None]


def _tc_dense(x_t, pos_chunk, gamma, beta, Wt, b_col, batch, seq_c, d, v,
              seq_total, pos_off, t_prev=None, interpret=False):
    dx = x_t.shape[1]
    grid = (seq_c, v // _VBLK)
    in_specs = [
        pl.BlockSpec((batch, dx), lambda s, vj: (s, 0)),
        pl.BlockSpec((1, 1, d), lambda s, vj: (s, 0, 0)),
        pl.BlockSpec((1, d), lambda s, vj: (0, 0)),
        pl.BlockSpec((1, d), lambda s, vj: (0, 0)),
        pl.BlockSpec((_VBLK, d), lambda s, vj: (vj, 0)),
        pl.BlockSpec((_VBLK, 1), lambda s, vj: (vj, 0)),
    ]
    args = [x_t, pos_chunk.reshape(seq_c, 1, d), gamma, beta, Wt, b_col]
    kwargs = {}
    body = _dense_body
    if t_prev is not None:
        in_specs.append(pl.BlockSpec(memory_space=pl.ANY))
        args.append(t_prev)
        kwargs["input_output_aliases"] = {6: 0}

        def body(x_ref, pos_ref, gamma_ref, beta_ref, wt_ref, b_ref,
                 _t_ref, o_ref, xn_ref):
            _dense_body(x_ref, pos_ref, gamma_ref, beta_ref, wt_ref,
                        b_ref, o_ref, xn_ref)

    return pl.pallas_call(
        body,
        grid=grid,
        in_specs=in_specs,
        out_specs=pl.BlockSpec((1, _VBLK, batch),
                               lambda s, vj: (pos_off + s, vj, 0)),
        out_shape=jax.ShapeDtypeStruct((seq_total, v, batch), jnp.float32),
        scratch_shapes=[pltpu.VMEM((batch, d), jnp.bfloat16)],
        compiler_params=pltpu.CompilerParams(
            dimension_semantics=("arbitrary", "arbitrary"),
        ),
        interpret=interpret,
        **kwargs,
    )(*args)


def kernel(inputs, tok_emb, pos_emb, gamma, beta, W, b):
    batch, seq = inputs.shape
    vocab, d = tok_emb.shape
    v_out = W.shape[1]

    n = batch * seq
    idx2d = inputs.T.reshape(1, n).astype(jnp.int32)
    d_pad = 128
    tok_pad = jnp.pad(tok_emb, ((0, 0), (0, d_pad - d)))

    seq_c = seq // _CHUNKS
    nc = batch * seq_c
    xs = [_sc_gather(tok_pad, idx2d[:, c * nc:(c + 1) * nc], nc, d_pad)
          for c in range(_CHUNKS)]
    t_buf = None
    for c in range(_CHUNKS):
        t_buf = _tc_dense(xs[c], pos_emb[c * seq_c:(c + 1) * seq_c],
                          gamma.reshape(1, d), beta.reshape(1, d),
                          W.T.astype(jnp.bfloat16),
                          b.reshape(v_out, 1), batch, seq_c, d, v_out,
                          seq, c * seq_c, t_prev=t_buf)
    return jnp.transpose(t_buf, (2, 0, 1))


# speedup vs baseline: 1.0936x; 1.0936x over previous
import jax
import jax.numpy as jnp
from jax.experimental import pallas as pl
from jax.experimental.pallas import tpu as pltpu
from jax.experimental.pallas import tpu_sc as plsc

EPS = 1e-3

_GATHER_WINDOW = 128
_VBLK = 1000
_CHUNKS = 2


def _sc_gather(tok_emb, idx2d, n, d):
    mesh = plsc.VectorSubcoreMesh(core_axis_name="core",
                                  subcore_axis_name="subcore")

    @pl.kernel(out_type=jax.ShapeDtypeStruct((n, d), tok_emb.dtype),
               mesh=mesh)
    def gather_kernel(x_hbm, i_hbm, o_hbm):
        def body(i_vmem, o_vmem):
            pltpu.sync_copy(x_hbm.at[i_vmem.at[0]], o_vmem)

        pltpu.emit_pipeline(
            body,
            grid=(n // _GATHER_WINDOW,),
            in_specs=[pl.BlockSpec((1, _GATHER_WINDOW),
                                   index_map=lambda i: (0, i))],
            out_specs=[pl.BlockSpec((_GATHER_WINDOW, d),
                                    index_map=lambda i: (i, 0))],
            core_axis_name=("core", "subcore"),
            dimension_semantics=(pltpu.PARALLEL,),
        )(i_hbm, o_hbm)

    return gather_kernel(tok_emb, idx2d)


def _dense_body(x_ref, pos_ref, gamma_ref, beta_ref, wt_ref, b_ref, o_ref,
                xn_ref):
    d = pos_ref.shape[2]
    vj = pl.program_id(1)

    @pl.when(vj == 0)
    def _():
        x = x_ref[:, :d] + pos_ref[0]
        mean = jnp.mean(x, axis=1, keepdims=True)
        xc = x - mean
        var = jnp.mean(xc * xc, axis=1, keepdims=True)
        xn = xc * jax.lax.rsqrt(var + EPS)
        xn = xn * gamma_ref[...] + beta_ref[...]
        xn_ref[...] = xn.astype(jnp.bfloat16)

    logits = jax.lax.dot_general(
        wt_ref[...], xn_ref[...],
        (((1,), (1,)), ((), ())),
        preferred_element_type=jnp.float32,
    ) + b_ref[...]
    o_ref[...] = logits[None]


def _tc_dense(x_t, pos_chunk, gamma, beta, Wt, b_col, batch, seq_c, d, v,
              seq_total, pos_off, t_prev=None, interpret=False):
    dx = x_t.shape[1]
    grid = (seq_c, v // _VBLK)
    in_specs = [
        pl.BlockSpec((batch, dx), lambda s, vj: (s, 0)),
        pl.BlockSpec((1, 1, d), lambda s, vj: (s, 0, 0)),
        pl.BlockSpec((1, d), lambda s, vj: (0, 0)),
        pl.BlockSpec((1, d), lambda s, vj: (0, 0)),
        pl.BlockSpec((_VBLK, d), lambda s, vj: (vj, 0)),
        pl.BlockSpec((_VBLK, 1), lambda s, vj: (vj, 0)),
    ]
    args = [x_t, pos_chunk.reshape(seq_c, 1, d), gamma, beta, Wt, b_col]
    kwargs = {}
    body = _dense_body
    if t_prev is not None:
        in_specs.append(pl.BlockSpec(memory_space=pl.ANY))
        args.append(t_prev)
        kwargs["input_output_aliases"] = {6: 0}

        def body(x_ref, pos_ref, gamma_ref, beta_ref, wt_ref, b_ref,
                 _t_ref, o_ref, xn_ref):
            _dense_body(x_ref, pos_ref, gamma_ref, beta_ref, wt_ref,
                        b_ref, o_ref, xn_ref)

    return pl.pallas_call(
        body,
        grid=grid,
        in_specs=in_specs,
        out_specs=pl.BlockSpec((1, _VBLK, batch),
                               lambda s, vj: (pos_off + s, vj, 0)),
        out_shape=jax.ShapeDtypeStruct((seq_total, v, batch), jnp.float32),
        scratch_shapes=[pltpu.VMEM((batch, d), jnp.bfloat16)],
        compiler_params=pltpu.CompilerParams(
            dimension_semantics=("arbitrary", "arbitrary"),
        ),
        interpret=interpret,
        **kwargs,
    )(*args)


def kernel(inputs, tok_emb, pos_emb, gamma, beta, W, b):
    batch, seq = inputs.shape
    vocab, d = tok_emb.shape
    v_out = W.shape[1]

    n = batch * seq
    idx2d = inputs.T.reshape(1, n).astype(jnp.int32)
    d_pad = 128
    tok_pad = jnp.pad(tok_emb, ((0, 0), (0, d_pad - d)))

    seq_c = seq // _CHUNKS
    nc = batch * seq_c
    xs = [_sc_gather(tok_pad, idx2d[:, c * nc:(c + 1) * nc], nc, d_pad)
          for c in range(_CHUNKS)]
    t_buf = None
    for c in range(_CHUNKS):
        t_buf = _tc_dense(xs[c], pos_emb[c * seq_c:(c + 1) * seq_c],
                          gamma.reshape(1, d), beta.reshape(1, d),
                          W.T.astype(jnp.bfloat16),
                          b.reshape(v_out, 1), batch, seq_c, d, v_out,
                          seq, c * seq_c, t_prev=t_buf)
    return jnp.transpose(t_buf, (2, 0, 1))


# speedup vs baseline: 1.1175x; 1.0219x over previous
import jax
import jax.numpy as jnp
from jax.experimental import pallas as pl
from jax.experimental.pallas import tpu as pltpu
from jax.experimental.pallas import tpu_sc as plsc

EPS = 1e-3

_GATHER_WINDOW = 128
_VBLK = 1000
_CHUNKS = 1


def _sc_gather(tok_emb, idx2d, n, d):
    mesh = plsc.VectorSubcoreMesh(core_axis_name="core",
                                  subcore_axis_name="subcore")

    @pl.kernel(out_type=jax.ShapeDtypeStruct((n, d), tok_emb.dtype),
               mesh=mesh)
    def gather_kernel(x_hbm, i_hbm, o_hbm):
        def body(i_vmem, o_vmem):
            pltpu.sync_copy(x_hbm.at[i_vmem.at[0]], o_vmem)

        pltpu.emit_pipeline(
            body,
            grid=(n // _GATHER_WINDOW,),
            in_specs=[pl.BlockSpec((1, _GATHER_WINDOW),
                                   index_map=lambda i: (0, i))],
            out_specs=[pl.BlockSpec((_GATHER_WINDOW, d),
                                    index_map=lambda i: (i, 0))],
            core_axis_name=("core", "subcore"),
            dimension_semantics=(pltpu.PARALLEL,),
        )(i_hbm, o_hbm)

    return gather_kernel(tok_emb, idx2d)


def _dense_body(x_ref, pos_ref, gamma_ref, beta_ref, wt_ref, b_ref, o_ref,
                xn_ref):
    d = pos_ref.shape[2]
    vj = pl.program_id(1)

    @pl.when(vj == 0)
    def _():
        x = x_ref[:, :d] + pos_ref[0]
        mean = jnp.mean(x, axis=1, keepdims=True)
        xc = x - mean
        var = jnp.mean(xc * xc, axis=1, keepdims=True)
        xn = xc * jax.lax.rsqrt(var + EPS)
        xn = xn * gamma_ref[...] + beta_ref[...]
        xn_ref[...] = xn.astype(jnp.bfloat16)

    logits = jax.lax.dot_general(
        wt_ref[...], xn_ref[...],
        (((1,), (1,)), ((), ())),
        preferred_element_type=jnp.float32,
    ) + b_ref[...]
    o_ref[...] = logits[None]


def _tc_dense(x_t, pos_chunk, gamma, beta, Wt, b_col, batch, seq_c, d, v,
              seq_total, pos_off, t_prev=None, interpret=False):
    dx = x_t.shape[1]
    grid = (seq_c, v // _VBLK)
    in_specs = [
        pl.BlockSpec((batch, dx), lambda s, vj: (s, 0)),
        pl.BlockSpec((1, 1, d), lambda s, vj: (s, 0, 0)),
        pl.BlockSpec((1, d), lambda s, vj: (0, 0)),
        pl.BlockSpec((1, d), lambda s, vj: (0, 0)),
        pl.BlockSpec((_VBLK, d), lambda s, vj: (vj, 0)),
        pl.BlockSpec((_VBLK, 1), lambda s, vj: (vj, 0)),
    ]
    args = [x_t, pos_chunk.reshape(seq_c, 1, d), gamma, beta, Wt, b_col]
    kwargs = {}
    body = _dense_body
    if t_prev is not None:
        in_specs.append(pl.BlockSpec(memory_space=pl.ANY))
        args.append(t_prev)
        kwargs["input_output_aliases"] = {6: 0}

        def body(x_ref, pos_ref, gamma_ref, beta_ref, wt_ref, b_ref,
                 _t_ref, o_ref, xn_ref):
            _dense_body(x_ref, pos_ref, gamma_ref, beta_ref, wt_ref,
                        b_ref, o_ref, xn_ref)

    return pl.pallas_call(
        body,
        grid=grid,
        in_specs=in_specs,
        out_specs=pl.BlockSpec((1, _VBLK, batch),
                               lambda s, vj: (pos_off + s, vj, 0)),
        out_shape=jax.ShapeDtypeStruct((seq_total, v, batch), jnp.float32),
        scratch_shapes=[pltpu.VMEM((batch, d), jnp.bfloat16)],
        compiler_params=pltpu.CompilerParams(
            dimension_semantics=("arbitrary", "arbitrary"),
        ),
        interpret=interpret,
        **kwargs,
    )(*args)


def kernel(inputs, tok_emb, pos_emb, gamma, beta, W, b):
    batch, seq = inputs.shape
    vocab, d = tok_emb.shape
    v_out = W.shape[1]

    n = batch * seq
    idx2d = inputs.T.reshape(1, n).astype(jnp.int32)
    d_pad = 128
    tok_pad = jnp.pad(tok_emb, ((0, 0), (0, d_pad - d)))

    seq_c = seq // _CHUNKS
    nc = batch * seq_c
    xs = [_sc_gather(tok_pad, idx2d[:, c * nc:(c + 1) * nc], nc, d_pad)
          for c in range(_CHUNKS)]
    t_buf = None
    for c in range(_CHUNKS):
        t_buf = _tc_dense(xs[c], pos_emb[c * seq_c:(c + 1) * seq_c],
                          gamma.reshape(1, d), beta.reshape(1, d),
                          W.T.astype(jnp.bfloat16),
                          b.reshape(v_out, 1), batch, seq_c, d, v_out,
                          seq, c * seq_c, t_prev=t_buf)
    return jnp.transpose(t_buf, (2, 0, 1))
